# baseline (speedup 1.0000x reference)
import functools

import numpy as np
import jax
import jax.numpy as jnp
from jax import lax
from jax.experimental import pallas as pl
from jax.experimental.pallas import tpu as pltpu
from jax.experimental.pallas import tpu_sc as plsc

B = 16384
R = 8
V = 64
NDIMS = 7
NC, NS, L = 2, 16, 16
NW = NC * NS
BPW = B // NW
GROUPS = BPW // L
SE = R + 1
SM = R * R + 1

T0_OFF = 0
MID_OFF = V * SE
T6_OFF = MID_OFF + 5 * V * SM
TAB_SIZE = T6_OFF + R * V


def _build_perm():
    src_off = [0, 512, 512 + 4096, 512 + 2 * 4096, 512 + 3 * 4096,
               512 + 4 * 4096, 512 + 5 * 4096]
    zero = 512 + 5 * 4096 + 512
    perm = np.full((TAB_SIZE,), zero, dtype=np.int32)
    for i in range(V):
        for j in range(R):
            perm[T0_OFF + i * SE + j] = src_off[0] + i * R + j
    for k in range(1, 6):
        for i in range(V):
            for j in range(R):
                for l in range(R):
                    perm[MID_OFF + (k - 1) * V * SM + i * SM + j * R + l] = (
                        src_off[k] + j * V * R + i * R + l)
    for j in range(R):
        for i in range(V):
            perm[T6_OFF + j * V + i] = src_off[6] + j * V + i
    return perm


_PERM = _build_perm()


def _tt_body(idx_hbm, tab_hbm, out_hbm, idx_v, tab_v, out_v, sem):
    wid = lax.axis_index("s") * NC + lax.axis_index("c")
    base = wid * BPW

    copies = [
        pltpu.async_copy(tab_hbm, tab_v, sem),
        pltpu.async_copy(idx_hbm.at[pl.ds(wid * (NDIMS * BPW), NDIMS * BPW)],
                         idx_v, sem),
    ]
    for c in copies:
        c.wait()

    @plsc.parallel_loop(0, GROUPS)
    def _group(g):
        o = g * L
        i0 = idx_v[pl.ds(0 * BPW + o, L)] * SE
        res = [plsc.load_gather(tab_v, [i0 + (T0_OFF + j)]) for j in range(R)]
        for k in range(1, 6):
            ik = (idx_v[pl.ds(k * BPW + o, L)] * SM
                  + (MID_OFF + (k - 1) * V * SM))
            new = []
            for l in range(R):
                acc = res[0] * plsc.load_gather(tab_v, [ik + l])
                for j in range(1, R):
                    acc = acc + res[j] * plsc.load_gather(
                        tab_v, [ik + (j * R + l)])
                new.append(acc)
            res = new
        i6 = idx_v[pl.ds(6 * BPW + o, L)]
        q = res[0] * plsc.load_gather(tab_v, [i6 + T6_OFF])
        for j in range(1, R):
            q = q + res[j] * plsc.load_gather(tab_v, [i6 + (T6_OFF + j * V)])
        out_v[pl.ds(o, L)] = q

    pltpu.sync_copy(out_v, out_hbm.at[pl.ds(base, BPW)])


_tt_gather = functools.partial(
    pl.kernel,
    out_type=jax.ShapeDtypeStruct((B,), jnp.float32),
    mesh=plsc.VectorSubcoreMesh(core_axis_name="c", subcore_axis_name="s",
                                num_cores=NC, num_subcores=NS),
    compiler_params=pltpu.CompilerParams(needs_layout_passes=False),
    scratch_types=[
        pltpu.VMEM((NDIMS * BPW,), jnp.int32),
        pltpu.VMEM((TAB_SIZE,), jnp.float32),
        pltpu.VMEM((BPW,), jnp.float32),
        pltpu.SemaphoreType.DMA,
    ],
)(_tt_body)


def kernel(states, actions, core0, core1, core2, core3, core4, core5, core6):
    idxp = (jnp.concatenate([states.T, actions.T], axis=0)
            .reshape(NDIMS, NW, BPW).transpose(1, 0, 2).reshape(-1))
    src = jnp.concatenate(
        [core0.reshape(-1), core1.reshape(-1), core2.reshape(-1),
         core3.reshape(-1), core4.reshape(-1), core5.reshape(-1),
         core6.reshape(-1), jnp.zeros((1,), jnp.float32)])
    tab = jnp.take(src, jnp.asarray(_PERM), axis=0)
    return _tt_gather(idxp, tab)


# speedup vs baseline: 1.1166x; 1.1166x over previous
import functools

import jax
import jax.numpy as jnp
from jax import lax
from jax.experimental import pallas as pl
from jax.experimental.pallas import tpu as pltpu
from jax.experimental.pallas import tpu_sc as plsc

B = 16384
R = 8
V = 64
NDIMS = 7
NC, NS, L = 2, 16, 16
NW = NC * NS
BPW = B // NW
GROUPS = BPW // L
SE = R + 1
SM = R * R + 1

RAW_MID = V * R
MID_T = V * SE
MIDSZ = V * SM


def _tt_body(idx_hbm, c0_hbm, c1_hbm, c2_hbm, c3_hbm, c4_hbm, c5_hbm,
             c6_hbm, out_hbm, idx_v, raw_v, tab_v, t6_v, out_v, sem):
    wid = lax.axis_index("s") * NC + lax.axis_index("c")
    base = wid * BPW

    copies = [
        pltpu.async_copy(c0_hbm, raw_v.at[pl.ds(0, V * R)], sem),
        pltpu.async_copy(c1_hbm, raw_v.at[pl.ds(RAW_MID, V * R * R)], sem),
        pltpu.async_copy(c2_hbm,
                         raw_v.at[pl.ds(RAW_MID + V * R * R, V * R * R)], sem),
        pltpu.async_copy(c3_hbm,
                         raw_v.at[pl.ds(RAW_MID + 2 * V * R * R, V * R * R)],
                         sem),
        pltpu.async_copy(c4_hbm,
                         raw_v.at[pl.ds(RAW_MID + 3 * V * R * R, V * R * R)],
                         sem),
        pltpu.async_copy(c5_hbm,
                         raw_v.at[pl.ds(RAW_MID + 4 * V * R * R, V * R * R)],
                         sem),
        pltpu.async_copy(c6_hbm, t6_v, sem),
        pltpu.async_copy(idx_hbm.at[pl.ds(wid * (NDIMS * BPW), NDIMS * BPW)],
                         idx_v, sem),
    ]
    for c in copies:
        c.wait()

    lane = lax.iota(jnp.int32, L)
    pat9 = lane + jnp.where(lane >= R, 1, 0)
    pat65 = lane + jnp.where(lane >= R, 57, 0)

    @plsc.parallel_loop(0, V // 2)
    def _repack(ip):
        x0 = raw_v[pl.ds(ip * (2 * R), 2 * R)]
        plsc.store_scatter(tab_v, [pat9 + ip * (2 * SE)], x0)
        for k in range(5):
            for j in range(R):
                x = raw_v[pl.ds(RAW_MID + k * (V * R * R) + j * (V * R)
                                + ip * (2 * R), 2 * R)]
                plsc.store_scatter(
                    tab_v,
                    [pat65 + (ip * (2 * SM) + (MID_T + k * MIDSZ + j * R))],
                    x)

    @plsc.parallel_loop(0, GROUPS)
    def _group(g):
        o = g * L
        i0 = idx_v[pl.ds(0 * BPW + o, L)] * SE
        res = [plsc.load_gather(tab_v, [i0 + j]) for j in range(R)]
        for k in range(1, 6):
            ik = (idx_v[pl.ds(k * BPW + o, L)] * SM
                  + (MID_T + (k - 1) * MIDSZ))
            new = []
            for l in range(R):
                acc = res[0] * plsc.load_gather(tab_v, [ik + l])
                for j in range(1, R):
                    acc = acc + res[j] * plsc.load_gather(
                        tab_v, [ik + (j * R + l)])
                new.append(acc)
            res = new
        i6 = idx_v[pl.ds(6 * BPW + o, L)]
        q = res[0] * plsc.load_gather(t6_v, [i6])
        for j in range(1, R):
            q = q + res[j] * plsc.load_gather(t6_v, [i6 + j * V])
        out_v[pl.ds(o, L)] = q

    pltpu.sync_copy(out_v, out_hbm.at[pl.ds(base, BPW)])


_tt_gather = functools.partial(
    pl.kernel,
    out_type=jax.ShapeDtypeStruct((B,), jnp.float32),
    mesh=plsc.VectorSubcoreMesh(core_axis_name="c", subcore_axis_name="s",
                                num_cores=NC, num_subcores=NS),
    compiler_params=pltpu.CompilerParams(needs_layout_passes=False),
    scratch_types=[
        pltpu.VMEM((NDIMS * BPW,), jnp.int32),
        pltpu.VMEM((RAW_MID + 5 * V * R * R,), jnp.float32),
        pltpu.VMEM((MID_T + 5 * MIDSZ,), jnp.float32),
        pltpu.VMEM((R * V,), jnp.float32),
        pltpu.VMEM((BPW,), jnp.float32),
        pltpu.SemaphoreType.DMA,
    ],
)(_tt_body)


def kernel(states, actions, core0, core1, core2, core3, core4, core5, core6):
    idxp = (jnp.concatenate([states.T, actions.T], axis=0)
            .reshape(NDIMS, NW, BPW).transpose(1, 0, 2).reshape(-1))
    return _tt_gather(idxp, core0.reshape(-1), core1.reshape(-1),
                      core2.reshape(-1), core3.reshape(-1),
                      core4.reshape(-1), core5.reshape(-1),
                      core6.reshape(-1))


# speedup vs baseline: 1.1641x; 1.0426x over previous
import functools

import jax
import jax.numpy as jnp
from jax import lax
from jax.experimental import pallas as pl
from jax.experimental.pallas import tpu as pltpu
from jax.experimental.pallas import tpu_sc as plsc

B = 16384
R = 8
V = 64
NDIMS = 7
NC, NS, L = 2, 16, 16
NW = NC * NS
BPW = B // NW
GROUPS = BPW // L
SE = R + 1
SM = R * R + 1


def _tt_body(idx_hbm, t0_hbm, t1_hbm, t2_hbm, t3_hbm, t4_hbm, t5_hbm,
             t6_hbm, out_hbm, idx_v, t0_v, t1_v, t2_v, t3_v, t4_v, t5_v,
             t6_v, out_v, sem):
    wid = lax.axis_index("s") * NC + lax.axis_index("c")
    base = wid * BPW

    copies = [
        pltpu.async_copy(t0_hbm, t0_v, sem),
        pltpu.async_copy(t1_hbm, t1_v, sem),
        pltpu.async_copy(t2_hbm, t2_v, sem),
        pltpu.async_copy(t3_hbm, t3_v, sem),
        pltpu.async_copy(t4_hbm, t4_v, sem),
        pltpu.async_copy(t5_hbm, t5_v, sem),
        pltpu.async_copy(t6_hbm, t6_v, sem),
        pltpu.async_copy(idx_hbm.at[pl.ds(wid * (NDIMS * BPW), NDIMS * BPW)],
                         idx_v, sem),
    ]
    for c in copies:
        c.wait()

    tmid = [t1_v, t2_v, t3_v, t4_v, t5_v]

    def group(g, carry):
        o = g * L
        i0 = idx_v[pl.ds(0 * BPW + o, L)] * SE
        res = [plsc.load_gather(t0_v, [i0 + j]) for j in range(R)]
        for k in range(1, 6):
            ik = idx_v[pl.ds(k * BPW + o, L)] * SM
            tk = tmid[k - 1]
            new = []
            for l in range(R):
                acc = res[0] * plsc.load_gather(tk, [ik + l])
                for j in range(1, R):
                    acc = acc + res[j] * plsc.load_gather(tk, [ik + (j * R + l)])
                new.append(acc)
            res = new
        i6 = idx_v[pl.ds(6 * BPW + o, L)] * SE
        q = res[0] * plsc.load_gather(t6_v, [i6 + 0])
        for j in range(1, R):
            q = q + res[j] * plsc.load_gather(t6_v, [i6 + j])
        out_v[pl.ds(o, L)] = q
        return carry

    lax.fori_loop(0, GROUPS, group, 0, unroll=False)
    pltpu.sync_copy(out_v, out_hbm.at[pl.ds(base, BPW)])


_tt_gather = functools.partial(
    pl.kernel,
    out_type=jax.ShapeDtypeStruct((B,), jnp.float32),
    mesh=plsc.VectorSubcoreMesh(core_axis_name="c", subcore_axis_name="s",
                                num_cores=NC, num_subcores=NS),
    compiler_params=pltpu.CompilerParams(needs_layout_passes=False),
    scratch_types=[
        pltpu.VMEM((NDIMS * BPW,), jnp.int32),
        pltpu.VMEM((V * SE,), jnp.float32),
        pltpu.VMEM((V * SM,), jnp.float32),
        pltpu.VMEM((V * SM,), jnp.float32),
        pltpu.VMEM((V * SM,), jnp.float32),
        pltpu.VMEM((V * SM,), jnp.float32),
        pltpu.VMEM((V * SM,), jnp.float32),
        pltpu.VMEM((V * SE,), jnp.float32),
        pltpu.VMEM((BPW,), jnp.float32),
        pltpu.SemaphoreType.DMA,
    ],
)(_tt_body)


def _pad_rows(t, stride):
    return jnp.pad(t, ((0, 0), (0, stride - t.shape[1]))).reshape(-1)


def kernel(states, actions, core0, core1, core2, core3, core4, core5, core6):
    idxp = (jnp.concatenate([states.T, actions.T], axis=0)
            .reshape(NDIMS, NW, BPW).transpose(1, 0, 2).reshape(-1))
    t0 = _pad_rows(core0.reshape(V, R), SE)
    tmid = [_pad_rows(jnp.transpose(c, (1, 0, 2)).reshape(V, R * R), SM)
            for c in (core1, core2, core3, core4, core5)]
    t6 = _pad_rows(jnp.transpose(core6, (1, 0, 2)).reshape(V, R), SE)
    return _tt_gather(idxp, t0, *tmid, t6)
